# baseline (speedup 1.0000x reference)
import jax
import jax.numpy as jnp
from jax import lax
from jax.experimental import pallas as pl
from jax.experimental.pallas import tpu as pltpu
from jax.experimental.pallas import tpu_sc as plsc

_N = 10000
_E = 320000
_D = 128
_H = 128
_DE = 16
_G = 64
_DCUT = 5.0

_NC = 2
_NS = 16
_NW = _NC * _NS
_K = 5
_CH = 40
_NPAD = 10240
_SLAB = _NPAD // _NS

_EHALF = _E // 2
_BE = 2000
_BN = 1000

_INTERPRET = False


def _elu(x):
    return jnp.where(x > 0, x, jnp.exp(x) - 1.0)


def _p1_body(sn, tn, wsT, wtT, sp, tp):
    sp[...] = jnp.dot(sn[...], wsT[...], preferred_element_type=jnp.float32)
    tp[...] = jnp.dot(tn[...], wtT[...], preferred_element_type=jnp.float32)


def _p1(source_node, target_node, wsT, wtT):
    grid = _N // _BN
    return pl.pallas_call(
        _p1_body,
        grid=(grid,),
        in_specs=[
            pl.BlockSpec((_BN, _D), lambda i: (i, 0)),
            pl.BlockSpec((_BN, _D), lambda i: (i, 0)),
            pl.BlockSpec((_D, _H), lambda i: (0, 0)),
            pl.BlockSpec((_D, _H), lambda i: (0, 0)),
        ],
        out_specs=[
            pl.BlockSpec((_BN, _H), lambda i: (i, 0)),
            pl.BlockSpec((_BN, _H), lambda i: (i, 0)),
        ],
        out_shape=[
            jax.ShapeDtypeStruct((_N, _H), jnp.float32),
            jax.ShapeDtypeStruct((_N, _H), jnp.float32),
        ],
        interpret=_INTERPRET,
    )(source_node, target_node, wsT, wtT)


def _sc_gather(src_proj, tgt_proj, i_src, i_tgt, eoff, ne):
    ew = ne // _NW
    ngrp = ew // _CH // _K

    def body(src_tab, tgt_tab, isrc, itgt, g_out, *rest):
        idx_s, idx_t, rows_s, rows_t = rest[:4]
        sem_i = rest[4]
        sem_g = rest[5:5 + _K]
        sem_o = rest[5 + _K:5 + 2 * _K]
        wid = lax.axis_index("s") * _NC + lax.axis_index("c")
        base = eoff + wid * ew

        hi = pltpu.async_copy(isrc.at[pl.ds(base, ew)], idx_s, sem_i)
        ht = pltpu.async_copy(itgt.at[pl.ds(base, ew)], idx_t, sem_i)
        hi.wait()
        ht.wait()

        def fire_gather(c, b):
            pltpu.async_copy(src_tab.at[idx_s.at[pl.ds(c * _CH, _CH)]], rows_s.at[b], sem_g[b])
            pltpu.async_copy(tgt_tab.at[idx_t.at[pl.ds(c * _CH, _CH)]], rows_t.at[b], sem_g[b])

        def wait_gather(b):
            pltpu.make_async_copy(g_out.at[pl.ds(0, _CH)], rows_s.at[b], sem_g[b]).wait()
            pltpu.make_async_copy(g_out.at[pl.ds(0, _CH)], rows_t.at[b], sem_g[b]).wait()

        def add_rows(b):
            def loop(r, carry):
                for u in range(2):
                    for j in range(_H // 16):
                        sl = pl.ds(j * 16, 16)
                        rows_s[b, 2 * r + u, sl] = rows_s[b, 2 * r + u, sl] + rows_t[b, 2 * r + u, sl]
                return carry
            lax.fori_loop(0, _CH // 2, loop, 0)

        def fire_store(c, b):
            off = wid * ew + c * _CH
            pltpu.async_copy(rows_s.at[b], g_out.at[pl.ds(off, _CH)], sem_o[b])

        def wait_store(b):
            pltpu.make_async_copy(rows_s.at[b], g_out.at[pl.ds(0, _CH)], sem_o[b]).wait()

        for b in range(_K):
            fire_gather(b, b)

        def group(g, carry):
            for b in range(_K):
                wait_gather(b)
                add_rows(b)
                fire_store(g * _K + b, b)
            for b in range(_K):
                wait_store(b)
                fire_gather((g + 1) * _K + b, b)
            return carry

        lax.fori_loop(0, ngrp - 1, group, 0)
        for b in range(_K):
            wait_gather(b)
            add_rows(b)
            fire_store((ngrp - 1) * _K + b, b)
        for b in range(_K):
            wait_store(b)

    mesh = plsc.VectorSubcoreMesh(core_axis_name="c", subcore_axis_name="s")
    f = pl.kernel(
        body,
        out_type=[jax.ShapeDtypeStruct((ne, _H), jnp.float32)],
        mesh=mesh,
        scratch_types=(
            [
                pltpu.VMEM((ew,), jnp.int32),
                pltpu.VMEM((ew,), jnp.int32),
                pltpu.VMEM((_K, _CH, _H), jnp.float32),
                pltpu.VMEM((_K, _CH, _H), jnp.float32),
                pltpu.SemaphoreType.DMA,
            ]
            + [pltpu.SemaphoreType.DMA] * (2 * _K)
        ),
    )
    return f(src_proj, tgt_proj, i_src, i_tgt)[0]


def _p3_body(g, ea, dist, bl, w, q, wdT, weaT, wm1T, bm1, wm2T, bm2, out):
    d = dist[...]
    d_c = jnp.minimum(d, _DCUT)
    x = d_c * (jnp.pi / _DCUT) - (0.5 * jnp.pi)
    x2 = x * x
    s = x * (1.0 + x2 * (-1.0 / 6.0 + x2 * (1.0 / 120.0 + x2 * (-1.0 / 5040.0
            + x2 * (1.0 / 362880.0 - x2 * (1.0 / 39916800.0))))))
    cutoff = 0.5 - 0.5 * s
    t = jnp.exp(-d)
    t2 = t * t
    rbf = cutoff * jnp.exp2(t2 * bl[...] + t * w[...] + q[...])
    dist_emb = jnp.tanh(jnp.dot(rbf, wdT[...], preferred_element_type=jnp.float32))
    lin = g[...] + jnp.dot(ea[...], weaT[...], preferred_element_type=jnp.float32)
    msg_in = dist_emb * lin
    h = _elu(jnp.dot(msg_in, wm1T[...], preferred_element_type=jnp.float32) + bm1[...])
    out[...] = _elu(jnp.dot(h, wm2T[...], preferred_element_type=jnp.float32) + bm2[...])


def _p3(g_rows, edge_attr, distance, bl2, w2, q2, wdT, weaT, wm1T, bm1, wm2T, bm2,
        eoff, ne):
    grid = ne // _BE
    ob = eoff // _BE
    return pl.pallas_call(
        _p3_body,
        grid=(grid,),
        in_specs=[
            pl.BlockSpec((_BE, _H), lambda i: (i, 0)),
            pl.BlockSpec((_BE, _DE), lambda i: (i + ob, 0)),
            pl.BlockSpec((_BE, 1), lambda i: (i + ob, 0)),
            pl.BlockSpec((1, _H), lambda i: (0, 0)),
            pl.BlockSpec((1, _H), lambda i: (0, 0)),
            pl.BlockSpec((1, _H), lambda i: (0, 0)),
            pl.BlockSpec((_H, _H), lambda i: (0, 0)),
            pl.BlockSpec((_DE, _H), lambda i: (0, 0)),
            pl.BlockSpec((_H, _H), lambda i: (0, 0)),
            pl.BlockSpec((1, _H), lambda i: (0, 0)),
            pl.BlockSpec((_H, _H), lambda i: (0, 0)),
            pl.BlockSpec((1, _H), lambda i: (0, 0)),
        ],
        out_specs=pl.BlockSpec((_BE, _H), lambda i: (i, 0)),
        out_shape=jax.ShapeDtypeStruct((ne, _H), jnp.float32),
        interpret=_INTERPRET,
    )(g_rows, edge_attr, distance, bl2, w2, q2, wdT, weaT, wm1T, bm1, wm2T, bm2)


def _sc_scatter(messages, i_tgt, zeros_hbm, eoff, ne):
    ew = ne // _NW
    ngrp = ew // _CH // _K

    def body(msgs, itgt, zeros, parts, *rest):
        idx_v = rest[:_K]
        accum, msg_v = rest[_K:_K + 2]
        sem_l = rest[_K + 2:_K + 2 + _K]
        sem_a = rest[_K + 2 + _K:]
        cid = lax.axis_index("c")
        sid = lax.axis_index("s")
        wid = sid * _NC + cid
        mbase = wid * ew
        ibase = eoff + wid * ew
        slab = sid * _SLAB

        pltpu.sync_copy(zeros.at[pl.ds(slab, _SLAB)], accum.at[pl.ds(slab, _SLAB)])
        plsc.subcore_barrier()

        def fire_load(c, b):
            pltpu.async_copy(itgt.at[pl.ds(ibase + c * _CH, _CH)], idx_v[b], sem_l[b])
            pltpu.async_copy(msgs.at[pl.ds(mbase + c * _CH, _CH)], msg_v.at[b], sem_l[b])

        def wait_load(b):
            pltpu.make_async_copy(itgt.at[pl.ds(0, _CH)], idx_v[b], sem_l[b]).wait()
            pltpu.make_async_copy(msgs.at[pl.ds(0, _CH)], msg_v.at[b], sem_l[b]).wait()

        for b in range(_K):
            fire_load(b, b)

        def group(g, carry):
            hs = []
            for b in range(_K):
                wait_load(b)
                hs.append(pltpu.async_copy(msg_v.at[b], accum.at[idx_v[b]], sem_a[b], add=True))
            for b in range(_K):
                hs[b].wait()
                fire_load((g + 1) * _K + b, b)
            return carry

        lax.fori_loop(0, ngrp - 1, group, 0)
        hs = []
        for b in range(_K):
            wait_load(b)
            hs.append(pltpu.async_copy(msg_v.at[b], accum.at[idx_v[b]], sem_a[b], add=True))
        for h in hs:
            h.wait()
        plsc.subcore_barrier()
        pltpu.sync_copy(accum.at[pl.ds(slab, _SLAB)],
                        parts.at[cid].at[pl.ds(slab, _SLAB)])

    mesh = plsc.VectorSubcoreMesh(core_axis_name="c", subcore_axis_name="s")
    f = pl.kernel(
        body,
        out_type=[jax.ShapeDtypeStruct((_NC, _NPAD, _H), jnp.float32)],
        mesh=mesh,
        scratch_types=(
            [pltpu.VMEM((_CH,), jnp.int32)] * _K
            + [
                pltpu.VMEM_SHARED((_NPAD, _H), jnp.float32),
                pltpu.VMEM((_K, _CH, _H), jnp.float32),
            ]
            + [pltpu.SemaphoreType.DMA] * (2 * _K)
        ),
    )
    return f(messages, i_tgt, zeros_hbm)[0]


def _p5a_body(tn, parts0, parts1, tb, wnT, waT, bc, pre_out, stats):
    aggr = parts0[0] + parts0[1] + parts1[0] + parts1[1]
    pre = _elu(jnp.dot(tn[...], wnT[...], preferred_element_type=jnp.float32)
               + jnp.dot(aggr, waT[...], preferred_element_type=jnp.float32)
               + bc[...])
    pre_out[...] = pre
    oh = (tb[...] == lax.broadcasted_iota(jnp.int32, (_BN, _G), 1)).astype(jnp.float32)
    ones = jnp.ones((_BN, _H), jnp.float32)
    dn = (((0,), (0,)), ((), ()))
    s0 = lax.dot_general(oh, ones, dn, preferred_element_type=jnp.float32)
    s1 = lax.dot_general(oh, pre, dn, preferred_element_type=jnp.float32)
    s2 = lax.dot_general(oh, pre * pre, dn, preferred_element_type=jnp.float32)
    contrib = jnp.concatenate([s0[None], s1[None], s2[None]], axis=0)

    @pl.when(pl.program_id(0) == 0)
    def _():
        stats[...] = jnp.zeros_like(stats)

    stats[...] = stats[...] + contrib


def _p5a(target_node, parts0, parts1, tb2d, wnT, waT, bc):
    grid = _N // _BN
    return pl.pallas_call(
        _p5a_body,
        grid=(grid,),
        in_specs=[
            pl.BlockSpec((_BN, _D), lambda i: (i, 0)),
            pl.BlockSpec((_NC, _BN, _H), lambda i: (0, i, 0)),
            pl.BlockSpec((_NC, _BN, _H), lambda i: (0, i, 0)),
            pl.BlockSpec((_BN, 1), lambda i: (i, 0)),
            pl.BlockSpec((_D, _H), lambda i: (0, 0)),
            pl.BlockSpec((_H, _H), lambda i: (0, 0)),
            pl.BlockSpec((1, _H), lambda i: (0, 0)),
        ],
        out_specs=[
            pl.BlockSpec((_BN, _H), lambda i: (i, 0)),
            pl.BlockSpec((3, _G, _H), lambda i: (0, 0, 0)),
        ],
        out_shape=[
            jax.ShapeDtypeStruct((_N, _H), jnp.float32),
            jax.ShapeDtypeStruct((3, _G, _H), jnp.float32),
        ],
        interpret=_INTERPRET,
    )(target_node, parts0, parts1, tb2d, wnT, waT, bc)


def _p5b_body(pre, stats, tb, gnw, gnb, gnms, out):
    c = jnp.maximum(stats[0], 1.0)
    mean = stats[1] / c
    a = gnms[...] * mean
    var = stats[2] / c - 2.0 * a * mean + a * a
    oh = (tb[...] == lax.broadcasted_iota(jnp.int32, (_BN, _G), 1)).astype(jnp.float32)
    a_rows = jnp.dot(oh, a, preferred_element_type=jnp.float32)
    v_rows = jnp.dot(oh, var, preferred_element_type=jnp.float32)
    out[...] = gnw[...] * (pre[...] - a_rows) * lax.rsqrt(v_rows + 1e-5) + gnb[...]


def _p5b(pre, stats, tb2d, gnw, gnb, gnms):
    grid = _N // _BN
    return pl.pallas_call(
        _p5b_body,
        grid=(grid,),
        in_specs=[
            pl.BlockSpec((_BN, _H), lambda i: (i, 0)),
            pl.BlockSpec((3, _G, _H), lambda i: (0, 0, 0)),
            pl.BlockSpec((_BN, 1), lambda i: (i, 0)),
            pl.BlockSpec((1, _H), lambda i: (0, 0)),
            pl.BlockSpec((1, _H), lambda i: (0, 0)),
            pl.BlockSpec((1, _H), lambda i: (0, 0)),
        ],
        out_specs=pl.BlockSpec((_BN, _H), lambda i: (i, 0)),
        out_shape=jax.ShapeDtypeStruct((_N, _H), jnp.float32),
        interpret=_INTERPRET,
    )(pre, stats, tb2d, gnw, gnb, gnms)


def kernel(source_node, target_node, edge_index, edge_attr, distance, target_batch,
           means, betas, W_dist, W_edge, W_m1, b_m1, W_m2, b_m2, W_res, W_comb,
           b_comb, gn_weight, gn_bias, gn_meanscale):
    i_src = edge_index[0]
    i_tgt = edge_index[1]
    wsT = W_edge[:, :_D].T
    wtT = W_edge[:, _D:2 * _D].T
    weaT = W_edge[:, 2 * _D:].T
    wnT = (W_res + W_comb[:, :_D]).T
    waT = W_comb[:, _D:].T
    log2e = jnp.float32(1.4426950408889634)
    bl2 = (-betas * log2e).reshape(1, _H)
    w2 = (2.0 * betas * means * log2e).reshape(1, _H)
    q2 = (-betas * means * means * log2e).reshape(1, _H)
    bm1 = b_m1.reshape(1, _H)
    bm2 = b_m2.reshape(1, _H)
    bc = b_comb.reshape(1, _H)
    gnw = gn_weight.reshape(1, _H)
    gnb = gn_bias.reshape(1, _H)
    gnms = gn_meanscale.reshape(1, _H)
    tb2d = target_batch.reshape(_N, 1)

    src_proj, tgt_proj = _p1(source_node, target_node, wsT, wtT)
    zeros_hbm = jnp.zeros((_NPAD, _H), jnp.float32)

    g0 = _sc_gather(src_proj, tgt_proj, i_src, i_tgt, 0, _EHALF)
    m0 = _p3(g0, edge_attr, distance, bl2, w2, q2,
             W_dist.T, weaT, W_m1.T, bm1, W_m2.T, bm2, 0, _EHALF)
    g1 = _sc_gather(src_proj, tgt_proj, i_src, i_tgt, _EHALF, _EHALF)
    parts0 = _sc_scatter(m0, i_tgt, zeros_hbm, 0, _EHALF)
    m1 = _p3(g1, edge_attr, distance, bl2, w2, q2,
             W_dist.T, weaT, W_m1.T, bm1, W_m2.T, bm2, _EHALF, _EHALF)
    parts1 = _sc_scatter(m1, i_tgt, zeros_hbm, _EHALF, _EHALF)
    pre, stats = _p5a(target_node, parts0, parts1, tb2d, wnT, waT, bc)
    return _p5b(pre, stats, tb2d, gnw, gnb, gnms)


# speedup vs baseline: 1.0491x; 1.0491x over previous
import jax
import jax.numpy as jnp
from jax import lax
from jax.experimental import pallas as pl
from jax.experimental.pallas import tpu as pltpu
from jax.experimental.pallas import tpu_sc as plsc

_N = 10000
_E = 320000
_D = 128
_H = 128
_DE = 16
_G = 64
_DCUT = 5.0

_NC = 2
_NS = 16
_NW = _NC * _NS
_K = 5
_CH = 40
_NPAD = 10240
_SLAB = _NPAD // _NS

_EHALF = _E // 2
_BE = 4000
_BN = 1000

_INTERPRET = False


def _elu(x):
    return jnp.where(x > 0, x, jnp.exp(x) - 1.0)


def _p1_body(sn, tn, wsT, wtT, sp, tp):
    sp[...] = jnp.dot(sn[...], wsT[...], preferred_element_type=jnp.float32)
    tp[...] = jnp.dot(tn[...], wtT[...], preferred_element_type=jnp.float32)


def _p1(source_node, target_node, wsT, wtT):
    grid = _N // _BN
    return pl.pallas_call(
        _p1_body,
        grid=(grid,),
        in_specs=[
            pl.BlockSpec((_BN, _D), lambda i: (i, 0)),
            pl.BlockSpec((_BN, _D), lambda i: (i, 0)),
            pl.BlockSpec((_D, _H), lambda i: (0, 0)),
            pl.BlockSpec((_D, _H), lambda i: (0, 0)),
        ],
        out_specs=[
            pl.BlockSpec((_BN, _H), lambda i: (i, 0)),
            pl.BlockSpec((_BN, _H), lambda i: (i, 0)),
        ],
        out_shape=[
            jax.ShapeDtypeStruct((_N, _H), jnp.float32),
            jax.ShapeDtypeStruct((_N, _H), jnp.float32),
        ],
        interpret=_INTERPRET,
    )(source_node, target_node, wsT, wtT)


def _sc_gather(src_proj, tgt_proj, i_src, i_tgt, eoff, ne):
    ew = ne // _NW
    ngrp = ew // _CH // _K

    def body(src_tab, tgt_tab, isrc, itgt, g_out, *rest):
        idx_s, idx_t, rows_s, rows_t = rest[:4]
        sem_i = rest[4]
        sem_g = rest[5:5 + _K]
        sem_o = rest[5 + _K:5 + 2 * _K]
        wid = lax.axis_index("s") * _NC + lax.axis_index("c")
        base = eoff + wid * ew

        hi = pltpu.async_copy(isrc.at[pl.ds(base, ew)], idx_s, sem_i)
        ht = pltpu.async_copy(itgt.at[pl.ds(base, ew)], idx_t, sem_i)
        hi.wait()
        ht.wait()

        def fire_gather(c, b):
            pltpu.async_copy(src_tab.at[idx_s.at[pl.ds(c * _CH, _CH)]], rows_s.at[b], sem_g[b])
            pltpu.async_copy(tgt_tab.at[idx_t.at[pl.ds(c * _CH, _CH)]], rows_t.at[b], sem_g[b])

        def wait_gather(b):
            pltpu.make_async_copy(g_out.at[pl.ds(0, _CH)], rows_s.at[b], sem_g[b]).wait()
            pltpu.make_async_copy(g_out.at[pl.ds(0, _CH)], rows_t.at[b], sem_g[b]).wait()

        def add_rows(b):
            def loop(r, carry):
                for u in range(2):
                    for j in range(_H // 16):
                        sl = pl.ds(j * 16, 16)
                        rows_s[b, 2 * r + u, sl] = rows_s[b, 2 * r + u, sl] + rows_t[b, 2 * r + u, sl]
                return carry
            lax.fori_loop(0, _CH // 2, loop, 0)

        def fire_store(c, b):
            off = wid * ew + c * _CH
            pltpu.async_copy(rows_s.at[b], g_out.at[pl.ds(off, _CH)], sem_o[b])

        def wait_store(b):
            pltpu.make_async_copy(rows_s.at[b], g_out.at[pl.ds(0, _CH)], sem_o[b]).wait()

        for b in range(_K):
            fire_gather(b, b)

        def group(g, carry):
            for b in range(_K):
                wait_gather(b)
                add_rows(b)
                fire_store(g * _K + b, b)
            for b in range(_K):
                wait_store(b)
                fire_gather((g + 1) * _K + b, b)
            return carry

        lax.fori_loop(0, ngrp - 1, group, 0)
        for b in range(_K):
            wait_gather(b)
            add_rows(b)
            fire_store((ngrp - 1) * _K + b, b)
        for b in range(_K):
            wait_store(b)

    mesh = plsc.VectorSubcoreMesh(core_axis_name="c", subcore_axis_name="s")
    f = pl.kernel(
        body,
        out_type=[jax.ShapeDtypeStruct((ne, _H), jnp.float32)],
        mesh=mesh,
        scratch_types=(
            [
                pltpu.VMEM((ew,), jnp.int32),
                pltpu.VMEM((ew,), jnp.int32),
                pltpu.VMEM((_K, _CH, _H), jnp.float32),
                pltpu.VMEM((_K, _CH, _H), jnp.float32),
                pltpu.SemaphoreType.DMA,
            ]
            + [pltpu.SemaphoreType.DMA] * (2 * _K)
        ),
    )
    return f(src_proj, tgt_proj, i_src, i_tgt)[0]


def _p3_body(g, ea, dist, bl, w, q, wdT, weaT, wm1T, bm1, wm2T, bm2, out):
    d = dist[...]
    d_c = jnp.minimum(d, _DCUT)
    x = d_c * (jnp.pi / _DCUT) - (0.5 * jnp.pi)
    x2 = x * x
    s = x * (1.0 + x2 * (-1.0 / 6.0 + x2 * (1.0 / 120.0 + x2 * (-1.0 / 5040.0
            + x2 * (1.0 / 362880.0 - x2 * (1.0 / 39916800.0))))))
    cutoff = 0.5 - 0.5 * s
    t = jnp.exp(-d)
    t2 = t * t
    rbf = cutoff * jnp.exp2(t2 * bl[...] + t * w[...] + q[...])
    dist_emb = jnp.tanh(jnp.dot(rbf, wdT[...], preferred_element_type=jnp.float32))
    lin = g[...] + jnp.dot(ea[...], weaT[...], preferred_element_type=jnp.float32)
    msg_in = dist_emb * lin
    h = _elu(jnp.dot(msg_in, wm1T[...], preferred_element_type=jnp.float32) + bm1[...])
    out[...] = _elu(jnp.dot(h, wm2T[...], preferred_element_type=jnp.float32) + bm2[...])


def _p3(g_rows, edge_attr, distance, bl2, w2, q2, wdT, weaT, wm1T, bm1, wm2T, bm2,
        eoff, ne):
    grid = ne // _BE
    ob = eoff // _BE
    return pl.pallas_call(
        _p3_body,
        grid=(grid,),
        in_specs=[
            pl.BlockSpec((_BE, _H), lambda i: (i, 0)),
            pl.BlockSpec((_BE, _DE), lambda i: (i + ob, 0)),
            pl.BlockSpec((_BE, 1), lambda i: (i + ob, 0)),
            pl.BlockSpec((1, _H), lambda i: (0, 0)),
            pl.BlockSpec((1, _H), lambda i: (0, 0)),
            pl.BlockSpec((1, _H), lambda i: (0, 0)),
            pl.BlockSpec((_H, _H), lambda i: (0, 0)),
            pl.BlockSpec((_DE, _H), lambda i: (0, 0)),
            pl.BlockSpec((_H, _H), lambda i: (0, 0)),
            pl.BlockSpec((1, _H), lambda i: (0, 0)),
            pl.BlockSpec((_H, _H), lambda i: (0, 0)),
            pl.BlockSpec((1, _H), lambda i: (0, 0)),
        ],
        out_specs=pl.BlockSpec((_BE, _H), lambda i: (i, 0)),
        out_shape=jax.ShapeDtypeStruct((ne, _H), jnp.float32),
        interpret=_INTERPRET,
    )(g_rows, edge_attr, distance, bl2, w2, q2, wdT, weaT, wm1T, bm1, wm2T, bm2)


def _sc_scatter(messages, i_tgt, zeros_hbm, eoff, ne):
    ew = ne // _NW
    ngrp = ew // _CH // _K

    def body(msgs, itgt, zeros, parts, *rest):
        idx_v = rest[:_K]
        accum, msg_v = rest[_K:_K + 2]
        sem_l = rest[_K + 2:_K + 2 + _K]
        sem_a = rest[_K + 2 + _K:]
        cid = lax.axis_index("c")
        sid = lax.axis_index("s")
        wid = sid * _NC + cid
        mbase = wid * ew
        ibase = eoff + wid * ew
        slab = sid * _SLAB

        pltpu.sync_copy(zeros.at[pl.ds(slab, _SLAB)], accum.at[pl.ds(slab, _SLAB)])
        plsc.subcore_barrier()

        def fire_load(c, b):
            pltpu.async_copy(itgt.at[pl.ds(ibase + c * _CH, _CH)], idx_v[b], sem_l[b])
            pltpu.async_copy(msgs.at[pl.ds(mbase + c * _CH, _CH)], msg_v.at[b], sem_l[b])

        def wait_load(b):
            pltpu.make_async_copy(itgt.at[pl.ds(0, _CH)], idx_v[b], sem_l[b]).wait()
            pltpu.make_async_copy(msgs.at[pl.ds(0, _CH)], msg_v.at[b], sem_l[b]).wait()

        for b in range(_K):
            fire_load(b, b)

        def group(g, carry):
            hs = []
            for b in range(_K):
                wait_load(b)
                hs.append(pltpu.async_copy(msg_v.at[b], accum.at[idx_v[b]], sem_a[b], add=True))
            for b in range(_K):
                hs[b].wait()
                fire_load((g + 1) * _K + b, b)
            return carry

        lax.fori_loop(0, ngrp - 1, group, 0)
        hs = []
        for b in range(_K):
            wait_load(b)
            hs.append(pltpu.async_copy(msg_v.at[b], accum.at[idx_v[b]], sem_a[b], add=True))
        for h in hs:
            h.wait()
        plsc.subcore_barrier()
        pltpu.sync_copy(accum.at[pl.ds(slab, _SLAB)],
                        parts.at[cid].at[pl.ds(slab, _SLAB)])

    mesh = plsc.VectorSubcoreMesh(core_axis_name="c", subcore_axis_name="s")
    f = pl.kernel(
        body,
        out_type=[jax.ShapeDtypeStruct((_NC, _NPAD, _H), jnp.float32)],
        mesh=mesh,
        scratch_types=(
            [pltpu.VMEM((_CH,), jnp.int32)] * _K
            + [
                pltpu.VMEM_SHARED((_NPAD, _H), jnp.float32),
                pltpu.VMEM((_K, _CH, _H), jnp.float32),
            ]
            + [pltpu.SemaphoreType.DMA] * (2 * _K)
        ),
    )
    return f(messages, i_tgt, zeros_hbm)[0]


def _p5a_body(tn, parts0, parts1, tb, wnT, waT, bc, pre_out, stats):
    aggr = parts0[0] + parts0[1] + parts1[0] + parts1[1]
    pre = _elu(jnp.dot(tn[...], wnT[...], preferred_element_type=jnp.float32)
               + jnp.dot(aggr, waT[...], preferred_element_type=jnp.float32)
               + bc[...])
    pre_out[...] = pre
    oh = (tb[...] == lax.broadcasted_iota(jnp.int32, (_BN, _G), 1)).astype(jnp.float32)
    ones = jnp.ones((_BN, _H), jnp.float32)
    dn = (((0,), (0,)), ((), ()))
    s0 = lax.dot_general(oh, ones, dn, preferred_element_type=jnp.float32)
    s1 = lax.dot_general(oh, pre, dn, preferred_element_type=jnp.float32)
    s2 = lax.dot_general(oh, pre * pre, dn, preferred_element_type=jnp.float32)
    contrib = jnp.concatenate([s0[None], s1[None], s2[None]], axis=0)

    @pl.when(pl.program_id(0) == 0)
    def _():
        stats[...] = jnp.zeros_like(stats)

    stats[...] = stats[...] + contrib


def _p5a(target_node, parts0, parts1, tb2d, wnT, waT, bc):
    grid = _N // _BN
    return pl.pallas_call(
        _p5a_body,
        grid=(grid,),
        in_specs=[
            pl.BlockSpec((_BN, _D), lambda i: (i, 0)),
            pl.BlockSpec((_NC, _BN, _H), lambda i: (0, i, 0)),
            pl.BlockSpec((_NC, _BN, _H), lambda i: (0, i, 0)),
            pl.BlockSpec((_BN, 1), lambda i: (i, 0)),
            pl.BlockSpec((_D, _H), lambda i: (0, 0)),
            pl.BlockSpec((_H, _H), lambda i: (0, 0)),
            pl.BlockSpec((1, _H), lambda i: (0, 0)),
        ],
        out_specs=[
            pl.BlockSpec((_BN, _H), lambda i: (i, 0)),
            pl.BlockSpec((3, _G, _H), lambda i: (0, 0, 0)),
        ],
        out_shape=[
            jax.ShapeDtypeStruct((_N, _H), jnp.float32),
            jax.ShapeDtypeStruct((3, _G, _H), jnp.float32),
        ],
        interpret=_INTERPRET,
    )(target_node, parts0, parts1, tb2d, wnT, waT, bc)


def _p5b_body(pre, stats, tb, gnw, gnb, gnms, out):
    c = jnp.maximum(stats[0], 1.0)
    mean = stats[1] / c
    a = gnms[...] * mean
    var = stats[2] / c - 2.0 * a * mean + a * a
    oh = (tb[...] == lax.broadcasted_iota(jnp.int32, (_BN, _G), 1)).astype(jnp.float32)
    a_rows = jnp.dot(oh, a, preferred_element_type=jnp.float32)
    v_rows = jnp.dot(oh, var, preferred_element_type=jnp.float32)
    out[...] = gnw[...] * (pre[...] - a_rows) * lax.rsqrt(v_rows + 1e-5) + gnb[...]


def _p5b(pre, stats, tb2d, gnw, gnb, gnms):
    grid = _N // _BN
    return pl.pallas_call(
        _p5b_body,
        grid=(grid,),
        in_specs=[
            pl.BlockSpec((_BN, _H), lambda i: (i, 0)),
            pl.BlockSpec((3, _G, _H), lambda i: (0, 0, 0)),
            pl.BlockSpec((_BN, 1), lambda i: (i, 0)),
            pl.BlockSpec((1, _H), lambda i: (0, 0)),
            pl.BlockSpec((1, _H), lambda i: (0, 0)),
            pl.BlockSpec((1, _H), lambda i: (0, 0)),
        ],
        out_specs=pl.BlockSpec((_BN, _H), lambda i: (i, 0)),
        out_shape=jax.ShapeDtypeStruct((_N, _H), jnp.float32),
        interpret=_INTERPRET,
    )(pre, stats, tb2d, gnw, gnb, gnms)


def kernel(source_node, target_node, edge_index, edge_attr, distance, target_batch,
           means, betas, W_dist, W_edge, W_m1, b_m1, W_m2, b_m2, W_res, W_comb,
           b_comb, gn_weight, gn_bias, gn_meanscale):
    i_src = edge_index[0]
    i_tgt = edge_index[1]
    wsT = W_edge[:, :_D].T
    wtT = W_edge[:, _D:2 * _D].T
    weaT = W_edge[:, 2 * _D:].T
    wnT = (W_res + W_comb[:, :_D]).T
    waT = W_comb[:, _D:].T
    log2e = jnp.float32(1.4426950408889634)
    bl2 = (-betas * log2e).reshape(1, _H)
    w2 = (2.0 * betas * means * log2e).reshape(1, _H)
    q2 = (-betas * means * means * log2e).reshape(1, _H)
    bm1 = b_m1.reshape(1, _H)
    bm2 = b_m2.reshape(1, _H)
    bc = b_comb.reshape(1, _H)
    gnw = gn_weight.reshape(1, _H)
    gnb = gn_bias.reshape(1, _H)
    gnms = gn_meanscale.reshape(1, _H)
    tb2d = target_batch.reshape(_N, 1)

    src_proj, tgt_proj = _p1(source_node, target_node, wsT, wtT)
    zeros_hbm = jnp.zeros((_NPAD, _H), jnp.float32)

    g0 = _sc_gather(src_proj, tgt_proj, i_src, i_tgt, 0, _EHALF)
    m0 = _p3(g0, edge_attr, distance, bl2, w2, q2,
             W_dist.T, weaT, W_m1.T, bm1, W_m2.T, bm2, 0, _EHALF)
    g1 = _sc_gather(src_proj, tgt_proj, i_src, i_tgt, _EHALF, _EHALF)
    parts0 = _sc_scatter(m0, i_tgt, zeros_hbm, 0, _EHALF)
    m1 = _p3(g1, edge_attr, distance, bl2, w2, q2,
             W_dist.T, weaT, W_m1.T, bm1, W_m2.T, bm2, _EHALF, _EHALF)
    parts1 = _sc_scatter(m1, i_tgt, zeros_hbm, _EHALF, _EHALF)
    pre, stats = _p5a(target_node, parts0, parts1, tb2d, wnT, waT, bc)
    return _p5b(pre, stats, tb2d, gnw, gnb, gnms)
